# baseline (device time: 536441 ns/iter reference)
import jax
import jax.numpy as jnp
from jax import lax
from jax.experimental import pallas as pl
from jax.experimental.pallas import tpu as pltpu

N_CHUNK = 8


def kernel(x):
    m, n = x.shape
    half_n = n // 2
    rows = m // N_CHUNK

    def body(x_ref, out_ref, copy_sem, send_sems, recv_sems):
        my_x = lax.axis_index("x")
        my_y = lax.axis_index("y")
        other_x = 1 - my_x

        barrier_sem = pltpu.get_barrier_semaphore()
        pl.semaphore_signal(
            barrier_sem, inc=1,
            device_id=(other_x, my_y), device_id_type=pl.DeviceIdType.MESH,
        )
        pl.semaphore_wait(barrier_sem, 1)

        rdmas = []
        for c in range(N_CHUNK):
            rdma = pltpu.make_async_remote_copy(
                src_ref=x_ref.at[
                    pl.ds(c * rows, rows), pl.ds(other_x * half_n, half_n)
                ],
                dst_ref=out_ref.at[pl.ds(my_x * m + c * rows, rows), :],
                send_sem=send_sems.at[c],
                recv_sem=recv_sems.at[c],
                device_id=(other_x, my_y),
                device_id_type=pl.DeviceIdType.MESH,
            )
            rdma.start()
            rdmas.append(rdma)

        local = pltpu.make_async_copy(
            x_ref.at[:, pl.ds(my_x * half_n, half_n)],
            out_ref.at[pl.ds(my_x * m, m), :],
            copy_sem,
        )
        local.start()
        local.wait()

        for rdma in rdmas:
            rdma.wait()

    return pl.pallas_call(
        body,
        out_shape=jax.ShapeDtypeStruct((2 * m, half_n), x.dtype),
        in_specs=[pl.BlockSpec(memory_space=pl.ANY)],
        out_specs=pl.BlockSpec(memory_space=pl.ANY),
        scratch_shapes=[
            pltpu.SemaphoreType.DMA,
            pltpu.SemaphoreType.DMA((N_CHUNK,)),
            pltpu.SemaphoreType.DMA((N_CHUNK,)),
        ],
        compiler_params=pltpu.CompilerParams(collective_id=0),
    )(x)


# device time: 536289 ns/iter; 1.0003x vs baseline; 1.0003x over previous
import jax
import jax.numpy as jnp
from jax import lax
from jax.experimental import pallas as pl
from jax.experimental.pallas import tpu as pltpu

N_CHUNK = 8


def kernel(x):
    m, n = x.shape
    half_n = n // 2
    rows = m // N_CHUNK

    def body(x_ref, out_ref, send_buf, copy_sem, fill_sems, send_sems, recv_sems):
        my_x = lax.axis_index("x")
        my_y = lax.axis_index("y")
        other_x = 1 - my_x

        barrier_sem = pltpu.get_barrier_semaphore()
        pl.semaphore_signal(
            barrier_sem, inc=1,
            device_id=(other_x, my_y), device_id_type=pl.DeviceIdType.MESH,
        )
        pl.semaphore_wait(barrier_sem, 1)

        local = pltpu.make_async_copy(
            x_ref.at[:, pl.ds(my_x * half_n, half_n)],
            out_ref.at[pl.ds(my_x * m, m), :],
            copy_sem,
        )
        local.start()

        def make_fill(c, slot):
            return pltpu.make_async_copy(
                x_ref.at[pl.ds(c * rows, rows), pl.ds(other_x * half_n, half_n)],
                send_buf.at[slot],
                fill_sems.at[slot],
            )

        def make_rdma(c, slot):
            return pltpu.make_async_remote_copy(
                src_ref=send_buf.at[slot],
                dst_ref=out_ref.at[pl.ds(my_x * m + c * rows, rows), :],
                send_sem=send_sems.at[slot],
                recv_sem=recv_sems.at[c],
                device_id=(other_x, my_y),
                device_id_type=pl.DeviceIdType.MESH,
            )

        rdmas = []
        make_fill(0, 0).start()
        for c in range(N_CHUNK):
            slot = c % 2
            make_fill(c, slot).wait()
            rdma = make_rdma(c, slot)
            rdma.start()
            rdmas.append(rdma)
            if c + 1 < N_CHUNK:
                if c >= 1:
                    rdmas[c - 1].wait_send()
                make_fill(c + 1, (c + 1) % 2).start()

        for rdma in rdmas[-2:]:
            rdma.wait_send()
        for rdma in rdmas:
            rdma.wait_recv()
        local.wait()

    return pl.pallas_call(
        body,
        out_shape=jax.ShapeDtypeStruct((2 * m, half_n), x.dtype),
        in_specs=[pl.BlockSpec(memory_space=pl.ANY)],
        out_specs=pl.BlockSpec(memory_space=pl.ANY),
        scratch_shapes=[
            pltpu.VMEM((2, rows, half_n), x.dtype),
            pltpu.SemaphoreType.DMA,
            pltpu.SemaphoreType.DMA((2,)),
            pltpu.SemaphoreType.DMA((2,)),
            pltpu.SemaphoreType.DMA((N_CHUNK,)),
        ],
        compiler_params=pltpu.CompilerParams(collective_id=0),
    )(x)
